# baseline (speedup 1.0000x reference)
import jax
import jax.numpy as jnp
from jax.experimental import pallas as pl
from jax.experimental.pallas import tpu as pltpu

POOL = 7
RB = 128
WN = 48


def _axis_coords(c, base, size):
    f32 = jnp.float32
    s = (jnp.arange(POOL, dtype=f32) + 0.5)[None, None, :] * (
        c.astype(f32) / POOL)[:, :, None] - 0.5
    s = jnp.clip(s, 0.0, (c.astype(f32) - 1.0)[:, :, None])
    i0 = jnp.floor(s).astype(jnp.int32)
    w = s - i0.astype(f32)
    cm1 = (c - 1)[:, :, None]
    a0 = jnp.clip(base[:, :, None] + jnp.clip(i0, 0, cm1), 0, size - 1)
    a1 = jnp.clip(base[:, :, None] + jnp.clip(i0 + 1, 0, cm1), 0, size - 1)
    return a0, a1, w


def _roi_kernel(ya0_ref, ya1_ref, wy_ref, xs_ref, fm_ref, k0_ref, out_ref):
    b = pl.program_id(0)
    rblk = pl.program_id(1)
    roi0 = (b * pl.num_programs(1) + rblk) * RB
    for rr in range(RB):
        k0r = k0_ref[0, rr]
        xs = pl.multiple_of(xs_ref[roi0 + rr], 16)
        for p in range(POOL):
            idx = (roi0 + rr) * POOL + p
            y0 = ya0_ref[idx]
            y1 = ya1_ref[idx]
            w = wy_ref[idx].astype(jnp.bfloat16)
            r0 = fm_ref[0, y0, pl.ds(xs, WN)]
            r1 = fm_ref[0, y1, pl.ds(xs, WN)]
            v = r0 + (r1 - r0) * w
            res = jax.lax.dot_general(
                k0r, v, (((1,), (0,)), ((), ())),
                preferred_element_type=jnp.float32)
            out_ref[0, p, :, rr, :] = res


@jax.jit
def kernel(feature_maps, rois):
    B, H, W, C = feature_maps.shape
    R = rois.shape[1]
    f32 = jnp.float32

    y1 = jnp.clip((rois[..., 0] * H).astype(jnp.int32), 0, H)
    x1 = jnp.clip((rois[..., 1] * W).astype(jnp.int32), 0, W)
    y2 = jnp.clip((rois[..., 2] * H).astype(jnp.int32), 0, H)
    x2 = jnp.clip((rois[..., 3] * W).astype(jnp.int32), 0, W)
    ch = jnp.maximum(y2 - y1, 1)
    cw = jnp.maximum(x2 - x1, 1)

    ya0, ya1, wy = _axis_coords(ch, y1, H)
    xa0, xa1, wx = _axis_coords(cw, x1, W)

    xs = jnp.minimum(x1 & ~15, W - WN)

    iota = jnp.arange(WN, dtype=jnp.int32)
    rel0 = (xa0 - xs[..., None])[..., None]
    rel1 = (xa1 - xs[..., None])[..., None]
    oh0 = (iota[None, None, None, :] == rel0).astype(f32)
    oh1 = (iota[None, None, None, :] == rel1).astype(f32)
    k0 = ((1.0 - wx)[..., None] * oh0 +
          wx[..., None] * oh1).astype(jnp.bfloat16)

    ya0_flat = ya0.reshape(-1)
    ya1_flat = ya1.reshape(-1)
    wy_flat = wy.reshape(-1)
    xs_flat = xs.reshape(-1)

    fm_bf16 = feature_maps.astype(jnp.bfloat16)

    grid_spec = pltpu.PrefetchScalarGridSpec(
        num_scalar_prefetch=4,
        grid=(B, R // RB),
        in_specs=[
            pl.BlockSpec((1, H, W, C), lambda b, r, *_: (b, 0, 0, 0)),
            pl.BlockSpec((1, RB, POOL, WN), lambda b, r, *_: (b, r, 0, 0)),
        ],
        out_specs=pl.BlockSpec((1, POOL, POOL, RB, C),
                               lambda b, r, *_: (b, 0, 0, r, 0)),
    )
    out = pl.pallas_call(
        _roi_kernel,
        out_shape=jax.ShapeDtypeStruct((B, POOL, POOL, R, C), f32),
        grid_spec=grid_spec,
        compiler_params=pltpu.CompilerParams(
            dimension_semantics=("parallel", "arbitrary"),
            vmem_limit_bytes=40 * 1024 * 1024,
        ),
        name="roi_pool_bilinear",
    )(ya0_flat, ya1_flat, wy_flat, xs_flat, fm_bf16, k0)
    return out.transpose(0, 3, 1, 2, 4)


# speedup vs baseline: 1.0236x; 1.0236x over previous
import jax
import jax.numpy as jnp
from jax.experimental import pallas as pl
from jax.experimental.pallas import tpu as pltpu

POOL = 7
RB = 64
WN = 48


def _axis_coords(c, base, size):
    f32 = jnp.float32
    s = (jnp.arange(POOL, dtype=f32) + 0.5)[None, None, :] * (
        c.astype(f32) / POOL)[:, :, None] - 0.5
    s = jnp.clip(s, 0.0, (c.astype(f32) - 1.0)[:, :, None])
    i0 = jnp.floor(s).astype(jnp.int32)
    w = s - i0.astype(f32)
    cm1 = (c - 1)[:, :, None]
    a0 = jnp.clip(base[:, :, None] + jnp.clip(i0, 0, cm1), 0, size - 1)
    a1 = jnp.clip(base[:, :, None] + jnp.clip(i0 + 1, 0, cm1), 0, size - 1)
    return a0, a1, w


def _roi_kernel(ya0_ref, ya1_ref, wy_ref, xs_ref, fm_ref, k0_ref, out_ref):
    b = pl.program_id(0)
    rblk = pl.program_id(1)
    roi0 = (b * pl.num_programs(1) + rblk) * RB
    for rr in range(RB):
        k0r = k0_ref[0, rr]
        xs = pl.multiple_of(xs_ref[roi0 + rr], 16)
        for p in range(POOL):
            idx = (roi0 + rr) * POOL + p
            y0 = ya0_ref[idx]
            y1 = ya1_ref[idx]
            w = wy_ref[idx].astype(jnp.bfloat16)
            r0 = fm_ref[0, y0, pl.ds(xs, WN)]
            r1 = fm_ref[0, y1, pl.ds(xs, WN)]
            v = r0 + (r1 - r0) * w
            res = jax.lax.dot_general(
                k0r, v, (((1,), (0,)), ((), ())),
                preferred_element_type=jnp.float32)
            out_ref[0, p, :, rr, :] = res


@jax.jit
def kernel(feature_maps, rois):
    B, H, W, C = feature_maps.shape
    R = rois.shape[1]
    f32 = jnp.float32

    y1 = jnp.clip((rois[..., 0] * H).astype(jnp.int32), 0, H)
    x1 = jnp.clip((rois[..., 1] * W).astype(jnp.int32), 0, W)
    y2 = jnp.clip((rois[..., 2] * H).astype(jnp.int32), 0, H)
    x2 = jnp.clip((rois[..., 3] * W).astype(jnp.int32), 0, W)
    ch = jnp.maximum(y2 - y1, 1)
    cw = jnp.maximum(x2 - x1, 1)

    ya0, ya1, wy = _axis_coords(ch, y1, H)
    xa0, xa1, wx = _axis_coords(cw, x1, W)

    xs = jnp.minimum(x1 & ~15, W - WN)

    iota = jnp.arange(WN, dtype=jnp.int32)
    rel0 = (xa0 - xs[..., None])[..., None]
    rel1 = (xa1 - xs[..., None])[..., None]
    oh0 = (iota[None, None, None, :] == rel0).astype(f32)
    oh1 = (iota[None, None, None, :] == rel1).astype(f32)
    k0 = ((1.0 - wx)[..., None] * oh0 +
          wx[..., None] * oh1).astype(jnp.bfloat16)

    ya0_flat = ya0.reshape(-1)
    ya1_flat = ya1.reshape(-1)
    wy_flat = wy.reshape(-1)
    xs_flat = xs.reshape(-1)

    fm_bf16 = feature_maps.astype(jnp.bfloat16)

    grid_spec = pltpu.PrefetchScalarGridSpec(
        num_scalar_prefetch=4,
        grid=(B, R // RB),
        in_specs=[
            pl.BlockSpec((1, H, W, C), lambda b, r, *_: (b, 0, 0, 0)),
            pl.BlockSpec((1, RB, POOL, WN), lambda b, r, *_: (b, r, 0, 0)),
        ],
        out_specs=pl.BlockSpec((1, POOL, POOL, RB, C),
                               lambda b, r, *_: (b, 0, 0, r, 0)),
    )
    out = pl.pallas_call(
        _roi_kernel,
        out_shape=jax.ShapeDtypeStruct((B, POOL, POOL, R, C), f32),
        grid_spec=grid_spec,
        compiler_params=pltpu.CompilerParams(
            dimension_semantics=("parallel", "arbitrary"),
            vmem_limit_bytes=40 * 1024 * 1024,
        ),
        name="roi_pool_bilinear",
    )(ya0_flat, ya1_flat, wy_flat, xs_flat, fm_bf16, k0)
    return out.transpose(0, 3, 1, 2, 4)


# speedup vs baseline: 1.1554x; 1.1287x over previous
import jax
import jax.numpy as jnp
from jax.experimental import pallas as pl
from jax.experimental.pallas import tpu as pltpu

POOL = 7
RB = 64
WN = 48


def _axis_coords(c, base, size):
    f32 = jnp.float32
    s = (jnp.arange(POOL, dtype=f32) + 0.5)[None, None, :] * (
        c.astype(f32) / POOL)[:, :, None] - 0.5
    s = jnp.clip(s, 0.0, (c.astype(f32) - 1.0)[:, :, None])
    i0 = jnp.floor(s).astype(jnp.int32)
    w = s - i0.astype(f32)
    cm1 = (c - 1)[:, :, None]
    a0 = jnp.clip(base[:, :, None] + jnp.clip(i0, 0, cm1), 0, size - 1)
    a1 = jnp.clip(base[:, :, None] + jnp.clip(i0 + 1, 0, cm1), 0, size - 1)
    return a0, a1, w


def _roi_kernel(ya0_ref, ya1_ref, wy_ref, xs_ref, fm_ref, k0_ref, out_ref,
                fmb_ref):
    b = pl.program_id(0)
    rblk = pl.program_id(1)

    @pl.when(rblk == 0)
    def _cast_fm():
        for h0 in range(0, fmb_ref.shape[0], 8):
            fmb_ref[h0:h0 + 8] = fm_ref[0, h0:h0 + 8].astype(jnp.bfloat16)

    roi0 = (b * pl.num_programs(1) + rblk) * RB
    for rr in range(RB):
        k0r = k0_ref[0, rr]
        xs = pl.multiple_of(xs_ref[roi0 + rr], 16)
        for p in range(POOL):
            idx = (roi0 + rr) * POOL + p
            y0 = ya0_ref[idx]
            y1 = ya1_ref[idx]
            w = wy_ref[idx].astype(jnp.bfloat16)
            r0 = fmb_ref[y0, pl.ds(xs, WN)]
            r1 = fmb_ref[y1, pl.ds(xs, WN)]
            v = r0 + (r1 - r0) * w
            res = jax.lax.dot_general(
                k0r, v, (((1,), (0,)), ((), ())),
                preferred_element_type=jnp.float32)
            out_ref[0, p, :, rr, :] = res


@jax.jit
def kernel(feature_maps, rois):
    B, H, W, C = feature_maps.shape
    R = rois.shape[1]
    f32 = jnp.float32

    y1 = jnp.clip((rois[..., 0] * H).astype(jnp.int32), 0, H)
    x1 = jnp.clip((rois[..., 1] * W).astype(jnp.int32), 0, W)
    y2 = jnp.clip((rois[..., 2] * H).astype(jnp.int32), 0, H)
    x2 = jnp.clip((rois[..., 3] * W).astype(jnp.int32), 0, W)
    ch = jnp.maximum(y2 - y1, 1)
    cw = jnp.maximum(x2 - x1, 1)

    ya0, ya1, wy = _axis_coords(ch, y1, H)
    xa0, xa1, wx = _axis_coords(cw, x1, W)

    xs = jnp.minimum(x1 & ~15, W - WN)

    iota = jnp.arange(WN, dtype=jnp.int32)
    rel0 = (xa0 - xs[..., None])[..., None]
    rel1 = (xa1 - xs[..., None])[..., None]
    oh0 = (iota[None, None, None, :] == rel0).astype(f32)
    oh1 = (iota[None, None, None, :] == rel1).astype(f32)
    k0 = ((1.0 - wx)[..., None] * oh0 +
          wx[..., None] * oh1).astype(jnp.bfloat16)

    ya0_flat = ya0.reshape(-1)
    ya1_flat = ya1.reshape(-1)
    wy_flat = wy.reshape(-1)
    xs_flat = xs.reshape(-1)

    grid_spec = pltpu.PrefetchScalarGridSpec(
        num_scalar_prefetch=4,
        grid=(B, R // RB),
        in_specs=[
            pl.BlockSpec((1, H, W, C), lambda b, r, *_: (b, 0, 0, 0)),
            pl.BlockSpec((1, RB, POOL, WN), lambda b, r, *_: (b, r, 0, 0)),
        ],
        out_specs=pl.BlockSpec((1, POOL, POOL, RB, C),
                               lambda b, r, *_: (b, 0, 0, r, 0)),
        scratch_shapes=[pltpu.VMEM((H, W, C), jnp.bfloat16)],
    )
    out = pl.pallas_call(
        _roi_kernel,
        out_shape=jax.ShapeDtypeStruct((B, POOL, POOL, R, C), f32),
        grid_spec=grid_spec,
        compiler_params=pltpu.CompilerParams(
            dimension_semantics=("parallel", "arbitrary"),
            vmem_limit_bytes=40 * 1024 * 1024,
        ),
        name="roi_pool_bilinear",
    )(ya0_flat, ya1_flat, wy_flat, xs_flat, feature_maps, k0)
    return out.transpose(0, 3, 1, 2, 4)
